# Initial kernel scaffold; baseline (speedup 1.0000x reference)
import jax
import jax.numpy as jnp
from jax.experimental import pallas as pl


def kernel(x, cluster_centers, filling_target, means_target, covs_target):
    raise NotImplementedError("write your pallas kernel here")


# speedup vs baseline: 29.1520x; 29.1520x over previous
import functools

import jax
import jax.numpy as jnp
from jax.experimental import pallas as pl
from jax.experimental.pallas import tpu as pltpu

_R = 256


def _mm(a, b):
    return jax.lax.dot_general(a, b, (((1,), (0,)), ((), ())),
                               preferred_element_type=jnp.float32)


def _body(n, d, k, r,
          x_ref, xt_ref, c_ref, ft_ref, mt_ref, ct_ref,
          loss_ref, s2_ref, cs_ref):
    i = pl.program_id(0)
    nsteps = pl.num_programs(0)

    x = x_ref[...]
    xt = xt_ref[...]
    c = c_ref[...]

    c2 = jnp.sum(c * c, axis=1, keepdims=True)
    d2t = c2 - 2.0 * _mm(c, xt)
    oh = (d2t == jnp.min(d2t, axis=0, keepdims=True)).astype(jnp.float32)

    col = jax.lax.broadcasted_iota(jnp.int32, (d, d * d), 1)
    row = jax.lax.broadcasted_iota(jnp.int32, (d, d * d), 0)
    e1 = ((col // d) == row).astype(jnp.float32)
    e2 = ((col % d) == row).astype(jnp.float32)
    mm_flat = lambda v: _mm(v, e1) * _mm(v, e2)
    y = mm_flat(x)
    xe = jnp.concatenate([x, jnp.ones((r, d), jnp.float32)], axis=1)

    s2_c = _mm(oh, y)
    cs_c = _mm(oh, xe)

    @pl.when(i == 0)
    def _():
        s2_ref[...] = s2_c
        cs_ref[...] = cs_c

    @pl.when(i > 0)
    def _():
        s2_ref[...] += s2_c
        cs_ref[...] += cs_c

    @pl.when(i == nsteps - 1)
    def _():
        s2 = s2_ref[...]
        sums = cs_ref[:, 0:d]
        counts = cs_ref[:, d:d + 1]

        filling = counts * (1.0 / n)
        loss_fil = jnp.sum((filling - ft_ref[...]) ** 2) * (1.0 / k)

        safe = jnp.maximum(counts, 1.0)
        inv = 1.0 / safe
        means = sums * inv
        loss_mean = jnp.sum((means - mt_ref[...]) ** 2) * (1.0 / (k * d))

        covs = s2 * inv - mm_flat(means)
        loss_cov = jnp.sum((covs - ct_ref[...]) ** 2) * (1.0 / (k * d * d))

        total = loss_fil + loss_mean + loss_cov
        loss_ref[...] = jnp.broadcast_to(total, (1, 1))


def kernel(x, cluster_centers, filling_target, means_target, covs_target):
    n, d = x.shape
    k = cluster_centers.shape[0]
    r = _R
    grid = (n // r,)

    out = pl.pallas_call(
        functools.partial(_body, n, d, k, r),
        grid=grid,
        in_specs=[
            pl.BlockSpec((r, d), lambda i: (i, 0)),
            pl.BlockSpec((d, r), lambda i: (0, i)),
            pl.BlockSpec((k, d), lambda i: (0, 0)),
            pl.BlockSpec((k, 1), lambda i: (0, 0)),
            pl.BlockSpec((k, d), lambda i: (0, 0)),
            pl.BlockSpec((k, d * d), lambda i: (0, 0)),
        ],
        out_specs=pl.BlockSpec((1, 1), lambda i: (0, 0)),
        out_shape=jax.ShapeDtypeStruct((1, 1), jnp.float32),
        scratch_shapes=[
            pltpu.VMEM((k, d * d), jnp.float32),
            pltpu.VMEM((k, 2 * d), jnp.float32),
        ],
        compiler_params=pltpu.CompilerParams(
            dimension_semantics=("arbitrary",),
        ),
    )(
        x,
        x.T,
        cluster_centers,
        filling_target.reshape(k, 1),
        means_target,
        covs_target.reshape(k, d * d),
    )
    return out[0, 0]


# speedup vs baseline: 52.1109x; 1.7876x over previous
import functools

import jax
import jax.numpy as jnp
from jax.experimental import pallas as pl
from jax.experimental.pallas import tpu as pltpu

_R = 4096


def _mm(a, b):
    return jax.lax.dot_general(a, b, (((1,), (0,)), ((), ())),
                               preferred_element_type=jnp.float32)


def _body(n, d, k, r,
          x_ref, xt_ref, c_ref, ft_ref, mt_ref, ct_ref,
          loss_ref, s2_ref, cs_ref):
    i = pl.program_id(0)
    nsteps = pl.num_programs(0)

    x = x_ref[...]
    xt = xt_ref[...]
    c = c_ref[...]

    c2 = jnp.sum(c * c, axis=1, keepdims=True)
    d2t = c2 - 2.0 * _mm(c, xt)
    oh = (d2t == jnp.min(d2t, axis=0, keepdims=True)).astype(jnp.bfloat16)

    col = jax.lax.broadcasted_iota(jnp.int32, (d, d * d), 1)
    row = jax.lax.broadcasted_iota(jnp.int32, (d, d * d), 0)
    e1 = ((col // d) == row).astype(jnp.bfloat16)
    e2 = ((col % d) == row).astype(jnp.bfloat16)
    x16 = x.astype(jnp.bfloat16)
    mm_flat = lambda v, ea, eb: (
        jax.lax.dot_general(v, ea, (((1,), (0,)), ((), ())),
                            preferred_element_type=jnp.float32)
        * jax.lax.dot_general(v, eb, (((1,), (0,)), ((), ())),
                              preferred_element_type=jnp.float32))
    a1 = jax.lax.dot_general(x16, e1, (((1,), (0,)), ((), ())),
                             preferred_element_type=jnp.float32)
    a2 = jnp.concatenate([x16] * d, axis=1)
    y = a1.astype(jnp.bfloat16) * a2
    xe = jnp.concatenate([x16, jnp.ones((r, d), jnp.bfloat16)], axis=1)

    s2_c = _mm(oh, y)
    cs_c = _mm(oh, xe)

    @pl.when(i == 0)
    def _():
        s2_ref[...] = s2_c
        cs_ref[...] = cs_c

    @pl.when(i > 0)
    def _():
        s2_ref[...] += s2_c
        cs_ref[...] += cs_c

    @pl.when(i == nsteps - 1)
    def _():
        s2 = s2_ref[...]
        sums = cs_ref[:, 0:d]
        counts = cs_ref[:, d:d + 1]

        filling = counts * (1.0 / n)
        loss_fil = jnp.sum((filling - ft_ref[...]) ** 2) * (1.0 / k)

        safe = jnp.maximum(counts, 1.0)
        inv = 1.0 / safe
        means = sums * inv
        loss_mean = jnp.sum((means - mt_ref[...]) ** 2) * (1.0 / (k * d))

        covs = s2 * inv - mm_flat(means.astype(jnp.bfloat16), e1, e2)
        loss_cov = jnp.sum((covs - ct_ref[...]) ** 2) * (1.0 / (k * d * d))

        total = loss_fil + loss_mean + loss_cov
        loss_ref[...] = jnp.broadcast_to(total, (1, 1))


def kernel(x, cluster_centers, filling_target, means_target, covs_target):
    n, d = x.shape
    k = cluster_centers.shape[0]
    r = _R
    grid = (n // r,)

    out = pl.pallas_call(
        functools.partial(_body, n, d, k, r),
        grid=grid,
        in_specs=[
            pl.BlockSpec((r, d), lambda i: (i, 0)),
            pl.BlockSpec((d, r), lambda i: (0, i)),
            pl.BlockSpec((k, d), lambda i: (0, 0)),
            pl.BlockSpec((k, 1), lambda i: (0, 0)),
            pl.BlockSpec((k, d), lambda i: (0, 0)),
            pl.BlockSpec((k, d * d), lambda i: (0, 0)),
        ],
        out_specs=pl.BlockSpec((1, 1), lambda i: (0, 0)),
        out_shape=jax.ShapeDtypeStruct((1, 1), jnp.float32),
        scratch_shapes=[
            pltpu.VMEM((k, d * d), jnp.float32),
            pltpu.VMEM((k, 2 * d), jnp.float32),
        ],
        compiler_params=pltpu.CompilerParams(
            dimension_semantics=("arbitrary",),
        ),
    )(
        x,
        x.T,
        cluster_centers,
        filling_target.reshape(k, 1),
        means_target,
        covs_target.reshape(k, d * d),
    )
    return out[0, 0]


# speedup vs baseline: 52.8582x; 1.0143x over previous
import functools

import jax
import jax.numpy as jnp
from jax.experimental import pallas as pl
from jax.experimental.pallas import tpu as pltpu

_R = 8192


def _mm(a, b):
    return jax.lax.dot_general(a, b, (((1,), (0,)), ((), ())),
                               preferred_element_type=jnp.float32)


def _body(n, d, k, r,
          x_ref, xt_ref, c_ref, ft_ref, mt_ref, ct_ref,
          loss_ref, s2_ref, cs_ref):
    i = pl.program_id(0)
    nsteps = pl.num_programs(0)

    x = x_ref[...]
    xt = xt_ref[...]
    c = c_ref[...]

    c2 = jnp.sum(c * c, axis=1, keepdims=True)
    d2t = c2 - 2.0 * _mm(c, xt)
    oh = (d2t == jnp.min(d2t, axis=0, keepdims=True)).astype(jnp.bfloat16)

    col = jax.lax.broadcasted_iota(jnp.int32, (d, d * d), 1)
    row = jax.lax.broadcasted_iota(jnp.int32, (d, d * d), 0)
    e1 = ((col // d) == row).astype(jnp.bfloat16)
    e2 = ((col % d) == row).astype(jnp.bfloat16)
    x16 = x.astype(jnp.bfloat16)
    mm_flat = lambda v, ea, eb: (
        jax.lax.dot_general(v, ea, (((1,), (0,)), ((), ())),
                            preferred_element_type=jnp.float32)
        * jax.lax.dot_general(v, eb, (((1,), (0,)), ((), ())),
                              preferred_element_type=jnp.float32))
    a2_4 = jnp.concatenate([x16] * 8, axis=1)
    gw = 8 * d
    ngrp = d // 8
    s2_chunks = []
    for g in range(ngrp):
        a1_g = jax.lax.dot_general(
            x16, e1[:, g * gw:(g + 1) * gw], (((1,), (0,)), ((), ())),
            preferred_element_type=jnp.float32)
        y_g = a1_g.astype(jnp.bfloat16) * a2_4
        s2_chunks.append(_mm(oh, y_g))
    xe = jnp.concatenate([x16, jnp.ones((r, d), jnp.bfloat16)], axis=1)
    cs_c = _mm(oh, xe)

    @pl.when(i == 0)
    def _():
        for g in range(ngrp):
            s2_ref[:, g * gw:(g + 1) * gw] = s2_chunks[g]
        cs_ref[...] = cs_c

    @pl.when(i > 0)
    def _():
        for g in range(ngrp):
            s2_ref[:, g * gw:(g + 1) * gw] += s2_chunks[g]
        cs_ref[...] += cs_c

    @pl.when(i == nsteps - 1)
    def _():
        s2 = s2_ref[...]
        sums = cs_ref[:, 0:d]
        counts = cs_ref[:, d:d + 1]

        filling = counts * (1.0 / n)
        loss_fil = jnp.sum((filling - ft_ref[...]) ** 2) * (1.0 / k)

        safe = jnp.maximum(counts, 1.0)
        inv = 1.0 / safe
        means = sums * inv
        loss_mean = jnp.sum((means - mt_ref[...]) ** 2) * (1.0 / (k * d))

        covs = s2 * inv - mm_flat(means.astype(jnp.bfloat16), e1, e2)
        loss_cov = jnp.sum((covs - ct_ref[...]) ** 2) * (1.0 / (k * d * d))

        total = loss_fil + loss_mean + loss_cov
        loss_ref[...] = jnp.broadcast_to(total, (1, 1))


def kernel(x, cluster_centers, filling_target, means_target, covs_target):
    n, d = x.shape
    k = cluster_centers.shape[0]
    r = _R
    grid = (n // r,)

    out = pl.pallas_call(
        functools.partial(_body, n, d, k, r),
        grid=grid,
        in_specs=[
            pl.BlockSpec((r, d), lambda i: (i, 0)),
            pl.BlockSpec((d, r), lambda i: (0, i)),
            pl.BlockSpec((k, d), lambda i: (0, 0)),
            pl.BlockSpec((k, 1), lambda i: (0, 0)),
            pl.BlockSpec((k, d), lambda i: (0, 0)),
            pl.BlockSpec((k, d * d), lambda i: (0, 0)),
        ],
        out_specs=pl.BlockSpec((1, 1), lambda i: (0, 0)),
        out_shape=jax.ShapeDtypeStruct((1, 1), jnp.float32),
        scratch_shapes=[
            pltpu.VMEM((k, d * d), jnp.float32),
            pltpu.VMEM((k, 2 * d), jnp.float32),
        ],
        compiler_params=pltpu.CompilerParams(
            dimension_semantics=("arbitrary",),
        ),
    )(
        x,
        x.T,
        cluster_centers,
        filling_target.reshape(k, 1),
        means_target,
        covs_target.reshape(k, d * d),
    )
    return out[0, 0]


# speedup vs baseline: 60.7412x; 1.1491x over previous
import functools

import numpy as np
import jax
import jax.numpy as jnp
from jax.experimental import pallas as pl
from jax.experimental.pallas import tpu as pltpu

_R = 8192


def _mm(a, b):
    return jax.lax.dot_general(a, b, (((1,), (0,)), ((), ())),
                               preferred_element_type=jnp.float32)


def _body(n, d, k, r,
          x_ref, xt_ref, c_ref, ft_ref, mt_ref, ct_ref,
          loss_ref, s2_ref, cs_ref):
    i = pl.program_id(0)
    nsteps = pl.num_programs(0)
    h = d // 2
    bw = h * h

    x = x_ref[...]
    xt = xt_ref[...]
    c = c_ref[...]

    c2 = jnp.sum(c * c, axis=1, keepdims=True)
    d2t = c2 - 2.0 * _mm(c, xt)
    oh = (d2t == jnp.min(d2t, axis=0, keepdims=True)).astype(jnp.bfloat16)

    colr = jax.lax.broadcasted_iota(jnp.int32, (d, bw), 1)
    rowr = jax.lax.broadcasted_iota(jnp.int32, (d, bw), 0)
    e_rep_lo = ((colr // h) == rowr).astype(jnp.bfloat16)
    e_rep_hi = ((colr // h + h) == rowr).astype(jnp.bfloat16)
    e_sel_lo = ((colr % h) == rowr).astype(jnp.bfloat16)
    e_sel_hi = ((colr % h + h) == rowr).astype(jnp.bfloat16)

    x16 = x.astype(jnp.bfloat16)
    t_lo = jnp.concatenate([x16[:, 0:h]] * h, axis=1)
    t_hi = jnp.concatenate([x16[:, h:d]] * h, axis=1)
    a_lo = _mm(x16, e_rep_lo).astype(jnp.bfloat16)
    a_hi = _mm(x16, e_rep_hi).astype(jnp.bfloat16)

    y_ll = a_lo * t_lo
    y_lu = a_lo * t_hi
    y_uu = a_hi * t_hi

    s2_ll = _mm(oh, y_ll)
    s2_lu = _mm(oh, y_lu)
    s2_uu = _mm(oh, y_uu)
    xe = jnp.concatenate([x16, jnp.ones((r, d), jnp.bfloat16)], axis=1)
    cs_c = _mm(oh, xe)

    @pl.when(i == 0)
    def _():
        s2_ref[:, 0 * bw:1 * bw] = s2_ll
        s2_ref[:, 1 * bw:2 * bw] = s2_lu
        s2_ref[:, 2 * bw:3 * bw] = s2_uu
        cs_ref[...] = cs_c

    @pl.when(i > 0)
    def _():
        s2_ref[:, 0 * bw:1 * bw] += s2_ll
        s2_ref[:, 1 * bw:2 * bw] += s2_lu
        s2_ref[:, 2 * bw:3 * bw] += s2_uu
        cs_ref[...] += cs_c

    @pl.when(i == nsteps - 1)
    def _():
        sums = cs_ref[:, 0:d]
        counts = cs_ref[:, d:d + 1]

        filling = counts * (1.0 / n)
        loss_fil = jnp.sum((filling - ft_ref[...]) ** 2) * (1.0 / k)

        safe = jnp.maximum(counts, 1.0)
        inv = 1.0 / safe
        means = sums * inv
        loss_mean = jnp.sum((means - mt_ref[...]) ** 2) * (1.0 / (k * d))

        m16 = means.astype(jnp.bfloat16)
        ma_lo = _mm(m16, e_rep_lo)
        ma_hi = _mm(m16, e_rep_hi)
        ms_lo = _mm(m16, e_sel_lo)
        ms_hi = _mm(m16, e_sel_hi)
        cov_ll = s2_ref[:, 0 * bw:1 * bw] * inv - ma_lo * ms_lo
        cov_lu = s2_ref[:, 1 * bw:2 * bw] * inv - ma_lo * ms_hi
        cov_uu = s2_ref[:, 2 * bw:3 * bw] * inv - ma_hi * ms_hi
        pr = jax.lax.broadcasted_iota(jnp.int32, (bw, bw), 0)
        pc = jax.lax.broadcasted_iota(jnp.int32, (bw, bw), 1)
        perm = (((pr // h) == (pc % h)) & ((pr % h) == (pc // h))
                ).astype(jnp.float32)
        cov_ul = _mm(cov_lu, perm)

        loss_cov = (jnp.sum((cov_ll - ct_ref[:, 0 * bw:1 * bw]) ** 2)
                    + jnp.sum((cov_lu - ct_ref[:, 1 * bw:2 * bw]) ** 2)
                    + jnp.sum((cov_uu - ct_ref[:, 2 * bw:3 * bw]) ** 2)
                    + jnp.sum((cov_ul - ct_ref[:, 3 * bw:4 * bw]) ** 2)
                    ) * (1.0 / (k * d * d))

        total = loss_fil + loss_mean + loss_cov
        loss_ref[...] = jnp.broadcast_to(total, (1, 1))


def _block_perm(d):
    h = d // 2
    a, b = np.meshgrid(np.arange(d), np.arange(d), indexing="ij")
    flat = (a * d + b).ravel()
    a, b = a.ravel(), b.ravel()

    def order(mask):
        idx = flat[mask]
        aa, bb = idx // d, idx % d
        return idx[np.argsort((aa % h) * h + (bb % h))]

    return np.concatenate([
        order((a < h) & (b < h)),
        order((a < h) & (b >= h)),
        order((a >= h) & (b >= h)),
        order((a >= h) & (b < h)),
    ])


def kernel(x, cluster_centers, filling_target, means_target, covs_target):
    n, d = x.shape
    k = cluster_centers.shape[0]
    r = _R
    grid = (n // r,)
    nb = 3 * (d // 2) * (d // 2)

    ct_blocks = covs_target.reshape(k, d * d)[:, _block_perm(d)]

    out = pl.pallas_call(
        functools.partial(_body, n, d, k, r),
        grid=grid,
        in_specs=[
            pl.BlockSpec((r, d), lambda i: (i, 0)),
            pl.BlockSpec((d, r), lambda i: (0, i)),
            pl.BlockSpec((k, d), lambda i: (0, 0)),
            pl.BlockSpec((k, 1), lambda i: (0, 0)),
            pl.BlockSpec((k, d), lambda i: (0, 0)),
            pl.BlockSpec((k, d * d), lambda i: (0, 0)),
        ],
        out_specs=pl.BlockSpec((1, 1), lambda i: (0, 0)),
        out_shape=jax.ShapeDtypeStruct((1, 1), jnp.float32),
        scratch_shapes=[
            pltpu.VMEM((k, nb), jnp.float32),
            pltpu.VMEM((k, 2 * d), jnp.float32),
        ],
        compiler_params=pltpu.CompilerParams(
            dimension_semantics=("arbitrary",),
            vmem_limit_bytes=65536 * 1024,
        ),
    )(
        x,
        x.T,
        cluster_centers,
        filling_target.reshape(k, 1),
        means_target,
        ct_blocks,
    )
    return out[0, 0]


# speedup vs baseline: 66.6212x; 1.0968x over previous
import functools

import numpy as np
import jax
import jax.numpy as jnp
from jax.experimental import pallas as pl
from jax.experimental.pallas import tpu as pltpu

_R = 8192


def _mm(a, b):
    return jax.lax.dot_general(a, b, (((1,), (0,)), ((), ())),
                               preferred_element_type=jnp.float32)


def _tile_lanes(u, w):
    t = u
    while t.shape[1] < w:
        t = jnp.concatenate([t, t], axis=1)
    return t


def _body(n, d, k, r,
          x_ref, xt_ref, c_ref, ft_ref, mt_ref, ct_ref,
          loss_ref, s2_ref, cs_ref):
    i = pl.program_id(0)
    nsteps = pl.num_programs(0)
    h = d // 2
    bw = h * h

    x16 = x_ref[...]
    xt = xt_ref[...]
    c = c_ref[...]

    c2 = jnp.sum(c * c, axis=1, keepdims=True)
    d2t = c2 - 2.0 * _mm(c, xt)
    oh = (d2t == jnp.min(d2t, axis=0, keepdims=True)).astype(jnp.bfloat16)

    colr = jax.lax.broadcasted_iota(jnp.int32, (d, bw), 1)
    rowr = jax.lax.broadcasted_iota(jnp.int32, (d, bw), 0)
    e_rep_lo = ((colr // h) == rowr).astype(jnp.bfloat16)
    e_rep_hi = ((colr // h + h) == rowr).astype(jnp.bfloat16)
    e_sel_lo = ((colr % h) == rowr).astype(jnp.bfloat16)
    e_sel_hi = ((colr % h + h) == rowr).astype(jnp.bfloat16)

    t_lo = _tile_lanes(x16[:, 0:h], bw)
    t_hi = _tile_lanes(x16[:, h:d], bw)
    a_lo = _mm(x16, e_rep_lo).astype(jnp.bfloat16)
    a_hi = _mm(x16, e_rep_hi).astype(jnp.bfloat16)

    y_ll = a_lo * t_lo
    y_lu = a_lo * t_hi
    y_uu = a_hi * t_hi

    s2_ll = _mm(oh, y_ll)
    s2_lu = _mm(oh, y_lu)
    s2_uu = _mm(oh, y_uu)
    xe = jnp.concatenate([x16, jnp.ones((r, d), jnp.bfloat16)], axis=1)
    cs_c = _mm(oh, xe)

    @pl.when(i == 0)
    def _():
        s2_ref[:, 0 * bw:1 * bw] = s2_ll
        s2_ref[:, 1 * bw:2 * bw] = s2_lu
        s2_ref[:, 2 * bw:3 * bw] = s2_uu
        cs_ref[...] = cs_c

    @pl.when(i > 0)
    def _():
        s2_ref[:, 0 * bw:1 * bw] += s2_ll
        s2_ref[:, 1 * bw:2 * bw] += s2_lu
        s2_ref[:, 2 * bw:3 * bw] += s2_uu
        cs_ref[...] += cs_c

    @pl.when(i == nsteps - 1)
    def _():
        sums = cs_ref[:, 0:d]
        counts = cs_ref[:, d:d + 1]

        filling = counts * (1.0 / n)
        loss_fil = jnp.sum((filling - ft_ref[...]) ** 2) * (1.0 / k)

        safe = jnp.maximum(counts, 1.0)
        inv = 1.0 / safe
        means = sums * inv
        loss_mean = jnp.sum((means - mt_ref[...]) ** 2) * (1.0 / (k * d))

        m16 = means.astype(jnp.bfloat16)
        ma_lo = _mm(m16, e_rep_lo)
        ma_hi = _mm(m16, e_rep_hi)
        ms_lo = _mm(m16, e_sel_lo)
        ms_hi = _mm(m16, e_sel_hi)
        cov_ll = s2_ref[:, 0 * bw:1 * bw] * inv - ma_lo * ms_lo
        cov_lu = s2_ref[:, 1 * bw:2 * bw] * inv - ma_lo * ms_hi
        cov_uu = s2_ref[:, 2 * bw:3 * bw] * inv - ma_hi * ms_hi
        pr = jax.lax.broadcasted_iota(jnp.int32, (bw, bw), 0)
        pc = jax.lax.broadcasted_iota(jnp.int32, (bw, bw), 1)
        perm = (((pr // h) == (pc % h)) & ((pr % h) == (pc // h))
                ).astype(jnp.float32)
        cov_ul = _mm(cov_lu, perm)

        loss_cov = (jnp.sum((cov_ll - ct_ref[:, 0 * bw:1 * bw]) ** 2)
                    + jnp.sum((cov_lu - ct_ref[:, 1 * bw:2 * bw]) ** 2)
                    + jnp.sum((cov_uu - ct_ref[:, 2 * bw:3 * bw]) ** 2)
                    + jnp.sum((cov_ul - ct_ref[:, 3 * bw:4 * bw]) ** 2)
                    ) * (1.0 / (k * d * d))

        total = loss_fil + loss_mean + loss_cov
        loss_ref[...] = jnp.broadcast_to(total, (1, 1))


def _block_perm(d):
    h = d // 2
    a, b = np.meshgrid(np.arange(d), np.arange(d), indexing="ij")
    flat = (a * d + b).ravel()
    a, b = a.ravel(), b.ravel()

    def order(mask):
        idx = flat[mask]
        aa, bb = idx // d, idx % d
        return idx[np.argsort((aa % h) * h + (bb % h))]

    return np.concatenate([
        order((a < h) & (b < h)),
        order((a < h) & (b >= h)),
        order((a >= h) & (b >= h)),
        order((a >= h) & (b < h)),
    ])


def kernel(x, cluster_centers, filling_target, means_target, covs_target):
    n, d = x.shape
    k = cluster_centers.shape[0]
    r = _R
    grid = (n // r,)
    nb = 3 * (d // 2) * (d // 2)

    ct_blocks = covs_target.reshape(k, d * d)[:, _block_perm(d)]

    out = pl.pallas_call(
        functools.partial(_body, n, d, k, r),
        grid=grid,
        in_specs=[
            pl.BlockSpec((r, d), lambda i: (i, 0)),
            pl.BlockSpec((d, r), lambda i: (0, i)),
            pl.BlockSpec((k, d), lambda i: (0, 0)),
            pl.BlockSpec((k, 1), lambda i: (0, 0)),
            pl.BlockSpec((k, d), lambda i: (0, 0)),
            pl.BlockSpec((k, d * d), lambda i: (0, 0)),
        ],
        out_specs=pl.BlockSpec((1, 1), lambda i: (0, 0)),
        out_shape=jax.ShapeDtypeStruct((1, 1), jnp.float32),
        scratch_shapes=[
            pltpu.VMEM((k, nb), jnp.float32),
            pltpu.VMEM((k, 2 * d), jnp.float32),
        ],
        compiler_params=pltpu.CompilerParams(
            dimension_semantics=("arbitrary",),
            vmem_limit_bytes=65536 * 1024,
        ),
    )(
        x.astype(jnp.bfloat16),
        x.T,
        cluster_centers,
        filling_target.reshape(k, 1),
        means_target,
        ct_blocks,
    )
    return out[0, 0]
